# baseline (speedup 1.0000x reference)
import jax
import jax.numpy as jnp
from jax import lax
from jax.experimental import pallas as pl
from jax.experimental.pallas import tpu as pltpu
from jax.experimental.pallas import tpu_sc as plsc

N = 10000
D = 128
E = 320000
G = 64
NSPLITS = 9
NC = 2
NS = 16
NW = NC * NS
EPW = 10240
CHUNKS = EPW // 128
NPAD = 10240
RPT = NPAD // NS

_f32 = jnp.float32
_MESH = plsc.VectorSubcoreMesh(core_axis_name="c", subcore_axis_name="s")


def _deg_body(dst_hbm, deg_out, dstv, hist, tbuf, accb, partials):
    c = lax.axis_index("c")
    t = lax.axis_index("s")
    w = c * NS + t
    z16 = jnp.zeros((16,), _f32)
    ones16 = jnp.ones((16,), _f32)

    def zero_hist(i, _):
        hist[pl.ds(i * 16, 16)] = z16
        return _

    lax.fori_loop(0, NPAD // 16, zero_hist, None)
    pltpu.sync_copy(dst_hbm.at[pl.ds(w * EPW, EPW)], dstv)

    def scat(j, _):
        idx = dstv[pl.ds(j * 16, 16)]
        plsc.addupdate_scatter(hist, [idx], ones16)
        return _

    lax.fori_loop(0, EPW // 16, scat, None)
    pltpu.sync_copy(hist, partials.at[pl.ds(t * NPAD, NPAD)])
    plsc.subcore_barrier()

    def zero_accb(i, _):
        accb[pl.ds(i * 16, 16)] = z16
        return _

    lax.fori_loop(0, RPT // 16, zero_accb, None)
    for tt in range(NS):
        pltpu.sync_copy(partials.at[pl.ds(tt * NPAD + t * RPT, RPT)], tbuf)

        def addv(i, _):
            accb[pl.ds(i * 16, 16)] = accb[pl.ds(i * 16, 16)] + tbuf[pl.ds(i * 16, 16)]
            return _

        lax.fori_loop(0, RPT // 16, addv, None)
    pltpu.sync_copy(accb, deg_out.at[pl.ds(c * NPAD + t * RPT, RPT)])


_deg_call = pl.kernel(
    _deg_body,
    out_type=jax.ShapeDtypeStruct((NC * NPAD,), _f32),
    mesh=_MESH,
    compiler_params=pltpu.CompilerParams(needs_layout_passes=False),
    scratch_types=[
        pltpu.VMEM((EPW,), jnp.int32),
        pltpu.VMEM((NPAD,), _f32),
        pltpu.VMEM((RPT,), _f32),
        pltpu.VMEM((RPT,), _f32),
        pltpu.VMEM_SHARED((NS * NPAD,), _f32),
    ],
)


TROWS = N // NC
MPAD = 10112
MRPT = MPAD // NS
EPT = 20480
CH = 64
_NSUP = EPT // 128


def _mp_body(u_hbm, src_hbm, dst_hbm, z_hbm, out_hbm,
             idxs, idxd, gbuf, dbuf, rows, table, acc, semi):
    c = lax.axis_index("c")
    t = lax.axis_index("s")
    base = t * EPT
    lo0 = c * TROWS

    def fire_idx(sup, s):
        off = base + sup * 128
        pltpu.async_copy(src_hbm.at[pl.ds(off, 128)], idxs[s], semi)
        pltpu.async_copy(dst_hbm.at[pl.ds(off, 128)], idxd[s], semi)

    def wait_idx(sup, s):
        off = base + sup * 128
        pltpu.make_async_copy(src_hbm.at[pl.ds(off, 128)], idxs[s],
                              semi).wait()
        pltpu.make_async_copy(dst_hbm.at[pl.ds(off, 128)], idxd[s],
                              semi).wait()

    fire_idx(0, 0)
    fire_idx(1, 1)
    @pl.when(t < NS - 1)
    def _stage_main():
        pltpu.sync_copy(u_hbm.at[pl.ds(lo0 + t * 312, 312), :],
                        table.at[pl.ds(t * 312, 312), :])

    @pl.when(t == NS - 1)
    def _stage_tail():
        pltpu.sync_copy(u_hbm.at[pl.ds(lo0 + 312 * (NS - 1), 320), :],
                        table.at[pl.ds(312 * (NS - 1), 320), :])

    pltpu.sync_copy(z_hbm, acc.at[pl.ds(t * MRPT, MRPT), :])
    plsc.subcore_barrier()

    def body(i, _):
        for s in range(2):
            sup = 2 * i + s
            wait_idx(sup, s)
            for q in range(2):
                for j in range(4):
                    sl = pl.ds(q * CH + j * 16, 16)
                    sv = idxs[s][sl] - lo0
                    inh = (sv >= 0) & (sv < TROWS)
                    gbuf[pl.ds(j * 16, 16)] = jnp.where(inh, sv, 0)
                    dbuf[pl.ds(j * 16, 16)] = jnp.where(inh, idxd[s][sl],
                                                        jnp.int32(N))
                pltpu.sync_copy(table.at[gbuf], rows)
                pltpu.sync_copy(rows, acc.at[dbuf], add=True)

            @pl.when(sup + 2 < _NSUP)
            def _prefetch():
                fire_idx(sup + 2, s)
        return _

    lax.fori_loop(0, _NSUP // 2, body, None)
    plsc.subcore_barrier()

    @pl.when(t < NS - 1)
    def _copy_main():
        pltpu.sync_copy(acc.at[pl.ds(t * 624, 624), :],
                        out_hbm.at[pl.ds(c * N + t * 624, 624), :])

    @pl.when(t == NS - 1)
    def _copy_tail():
        pltpu.sync_copy(acc.at[pl.ds(624 * (NS - 1), 640), :],
                        out_hbm.at[pl.ds(c * N + 624 * (NS - 1), 640), :])


_mp_call = pl.kernel(
    _mp_body,
    out_type=jax.ShapeDtypeStruct((NC * N, D), _f32),
    mesh=_MESH,
    compiler_params=pltpu.CompilerParams(needs_layout_passes=False),
    scratch_types=[
        [pltpu.VMEM((128,), jnp.int32) for _ in range(2)],
        [pltpu.VMEM((128,), jnp.int32) for _ in range(2)],
        pltpu.VMEM((CH,), jnp.int32),
        pltpu.VMEM((CH,), jnp.int32),
        pltpu.VMEM((CH, D), _f32),
        pltpu.VMEM_SHARED((TROWS, D), _f32),
        pltpu.VMEM_SHARED((MPAD, D), _f32),
        pltpu.SemaphoreType.DMA,
    ],
)


def _prep_tc(deg0_ref, deg1_ref, x_ref, u0_ref, dinv_ref):
    dinv = lax.rsqrt(deg0_ref[...] + deg1_ref[...] + 1.0)
    u0_ref[...] = x_ref[...] * dinv
    dinv_ref[...] = dinv


_prep_call = pl.pallas_call(
    _prep_tc,
    out_shape=(jax.ShapeDtypeStruct((N, D), _f32),
               jax.ShapeDtypeStruct((N, 1), _f32)),
)


def _bn_relu_tc(zz, g, be):
    mu = jnp.mean(zz, axis=0, keepdims=True)
    dz = zz - mu
    var = jnp.mean(dz * dz, axis=0, keepdims=True)
    return jnp.maximum(dz * lax.rsqrt(var + 1e-5) * g + be, 0.0)


def _layer_tc(m_ref, u_ref, dinv_ref, w_ref, b_ref, g_ref, be_ref, un_ref):
    dinv = dinv_ref[...]
    s = (m_ref[0] + m_ref[1] + u_ref[...]) * dinv
    zz = jnp.dot(s, w_ref[...], preferred_element_type=_f32) + b_ref[...]
    h = _bn_relu_tc(zz, g_ref[...], be_ref[...])
    un_ref[...] = h * dinv


_layer_call = pl.pallas_call(
    _layer_tc,
    out_shape=jax.ShapeDtypeStruct((N, D), _f32),
)


def _heads_tc(m_ref, u_ref, dinv_ref, batch_ref, w_ref, b_ref, g_ref, be_ref,
              fcw_ref, fcb_ref, out_ref, pre_s, s_s, icnt_s):
    j = pl.program_id(0)

    @pl.when(j == 0)
    def _init():
        dinv = dinv_ref[...]
        pre_s[...] = (m_ref[0] + m_ref[1] + u_ref[...]) * dinv
        gid = lax.broadcasted_iota(jnp.int32, (G, N), 0)
        sel = (gid == batch_ref[...]).astype(_f32)
        s_s[...] = sel
        cnt = jnp.sum(sel, axis=1, keepdims=True)
        icnt_s[...] = 1.0 / jnp.maximum(cnt, 1.0)

    zz = jnp.dot(pre_s[...], w_ref[0], preferred_element_type=_f32) + b_ref[0]
    h = _bn_relu_tc(zz, g_ref[0], be_ref[0])
    sums = jnp.dot(s_s[...], h, preferred_element_type=_f32)
    pooled = sums * icnt_s[...]
    logits = jnp.dot(pooled, fcw_ref[0], preferred_element_type=_f32) + fcb_ref[0]
    mx = jnp.max(logits, axis=1, keepdims=True)
    ex = jnp.exp(logits - mx)
    out_ref[0] = (logits - mx) - jnp.log(jnp.sum(ex, axis=1, keepdims=True))


_heads_call = pl.pallas_call(
    _heads_tc,
    grid=(NSPLITS,),
    in_specs=[
        pl.BlockSpec((NC, N, D), lambda j: (0, 0, 0)),
        pl.BlockSpec((N, D), lambda j: (0, 0)),
        pl.BlockSpec((N, 1), lambda j: (0, 0)),
        pl.BlockSpec((1, N), lambda j: (0, 0)),
        pl.BlockSpec((1, D, D), lambda j: (j, 0, 0)),
        pl.BlockSpec((1, 1, D), lambda j: (j, 0, 0)),
        pl.BlockSpec((1, 1, D), lambda j: (j, 0, 0)),
        pl.BlockSpec((1, 1, D), lambda j: (j, 0, 0)),
        pl.BlockSpec((1, D, 10), lambda j: (j, 0, 0)),
        pl.BlockSpec((1, 1, 10), lambda j: (j, 0, 0)),
    ],
    out_specs=pl.BlockSpec((1, G, 10), lambda j: (j, 0, 0)),
    out_shape=jax.ShapeDtypeStruct((NSPLITS, G, 10), _f32),
    scratch_shapes=[
        pltpu.VMEM((N, D), _f32),
        pltpu.VMEM((G, N), _f32),
        pltpu.VMEM((G, 1), _f32),
    ],
)


def kernel(x, edge_index, batch, shared_W, shared_b, shared_g, shared_be,
           main_W, main_b, main_g, main_be, main_fcW, main_fcb,
           aux_W, aux_b, aux_g, aux_be, aux_fcW, aux_fcb):
    ept_raw = E // NS
    src = edge_index[0].reshape(NS, ept_raw)
    dst = edge_index[1].reshape(NS, ept_raw)
    srcp = jnp.pad(src, ((0, 0), (0, EPT - ept_raw))).reshape(-1)
    dstp = jnp.pad(dst, ((0, 0), (0, EPT - ept_raw)), constant_values=N).reshape(-1)
    z_rows = jnp.zeros((MRPT, D), _f32)

    deg = _deg_call(dstp)
    deg0 = deg[:N].reshape(N, 1)
    deg1 = deg[NPAD:NPAD + N].reshape(N, 1)
    u0, dinv = _prep_call(deg0, deg1, x)

    m0 = _mp_call(u0, srcp, dstp, z_rows).reshape(NC, N, D)
    u1 = _layer_call(m0, u0, dinv, shared_W[0], shared_b[0].reshape(1, D),
                     shared_g[0].reshape(1, D), shared_be[0].reshape(1, D))
    m1 = _mp_call(u1, srcp, dstp, z_rows).reshape(NC, N, D)
    u2 = _layer_call(m1, u1, dinv, shared_W[1], shared_b[1].reshape(1, D),
                     shared_g[1].reshape(1, D), shared_be[1].reshape(1, D))
    m2 = _mp_call(u2, srcp, dstp, z_rows).reshape(NC, N, D)

    window = D // NSPLITS
    Ws = jnp.stack([main_W] + [jnp.roll(aux_W[i], window * (i + 1), axis=0)
                               for i in range(NSPLITS - 1)])
    bs = jnp.concatenate([main_b[None], aux_b]).reshape(NSPLITS, 1, D)
    gs = jnp.concatenate([main_g[None], aux_g]).reshape(NSPLITS, 1, D)
    bes = jnp.concatenate([main_be[None], aux_be]).reshape(NSPLITS, 1, D)
    fcWs = jnp.concatenate([main_fcW[None], aux_fcW])
    fcbs = jnp.concatenate([main_fcb[None], aux_fcb]).reshape(NSPLITS, 1, 10)

    outs = _heads_call(m2, u2, dinv, batch.reshape(1, N),
                       Ws, bs, gs, bes, fcWs, fcbs)
    return (outs[0], jnp.swapaxes(outs[1:], 0, 1))


# speedup vs baseline: 1.0913x; 1.0913x over previous
import jax
import jax.numpy as jnp
from jax import lax
from jax.experimental import pallas as pl
from jax.experimental.pallas import tpu as pltpu
from jax.experimental.pallas import tpu_sc as plsc

N = 10000
D = 128
E = 320000
G = 64
NSPLITS = 9
NC = 2
NS = 16
NW = NC * NS
EPW = 10240
CHUNKS = EPW // 128
NPAD = 10240
RPT = NPAD // NS

_f32 = jnp.float32
_MESH = plsc.VectorSubcoreMesh(core_axis_name="c", subcore_axis_name="s")


def _deg_body(dst_hbm, deg_out, dstv, hist, tbuf, accb, partials):
    c = lax.axis_index("c")
    t = lax.axis_index("s")
    w = c * NS + t
    z16 = jnp.zeros((16,), _f32)
    ones16 = jnp.ones((16,), _f32)

    def zero_hist(i, _):
        hist[pl.ds(i * 16, 16)] = z16
        return _

    lax.fori_loop(0, NPAD // 16, zero_hist, None)
    pltpu.sync_copy(dst_hbm.at[pl.ds(w * EPW, EPW)], dstv)

    def scat(j, _):
        idx = dstv[pl.ds(j * 16, 16)]
        plsc.addupdate_scatter(hist, [idx], ones16)
        return _

    lax.fori_loop(0, EPW // 16, scat, None)
    pltpu.sync_copy(hist, partials.at[pl.ds(t * NPAD, NPAD)])
    plsc.subcore_barrier()

    def zero_accb(i, _):
        accb[pl.ds(i * 16, 16)] = z16
        return _

    lax.fori_loop(0, RPT // 16, zero_accb, None)
    for tt in range(NS):
        pltpu.sync_copy(partials.at[pl.ds(tt * NPAD + t * RPT, RPT)], tbuf)

        def addv(i, _):
            accb[pl.ds(i * 16, 16)] = accb[pl.ds(i * 16, 16)] + tbuf[pl.ds(i * 16, 16)]
            return _

        lax.fori_loop(0, RPT // 16, addv, None)
    pltpu.sync_copy(accb, deg_out.at[pl.ds(c * NPAD + t * RPT, RPT)])


_deg_call = pl.kernel(
    _deg_body,
    out_type=jax.ShapeDtypeStruct((NC * NPAD,), _f32),
    mesh=_MESH,
    compiler_params=pltpu.CompilerParams(needs_layout_passes=False),
    scratch_types=[
        pltpu.VMEM((EPW,), jnp.int32),
        pltpu.VMEM((NPAD,), _f32),
        pltpu.VMEM((RPT,), _f32),
        pltpu.VMEM((RPT,), _f32),
        pltpu.VMEM_SHARED((NS * NPAD,), _f32),
    ],
)


CH = 64
NCHUNK = EPW // CH
_NBUF = 5
_NSET = 10
_LA = 3
_UNROLL = 10


def _mp_body(u_hbm, src_hbm, dst_hbm, z_hbm, out_hbm,
             idxs, idxd, rows, acc, semi, semg, sems):
    c = lax.axis_index("c")
    t = lax.axis_index("s")
    base = (c * NS + t) * EPW

    def fire_idx(k, off):
        pltpu.async_copy(src_hbm.at[pl.ds(off, CH)], idxs[k % _NSET], semi)
        pltpu.async_copy(dst_hbm.at[pl.ds(off, CH)], idxd[k % _NSET], semi)

    def sync_idx(k):
        off = base + k * CH
        pltpu.sync_copy(src_hbm.at[pl.ds(off, CH)], idxs[k % _NSET])
        pltpu.sync_copy(dst_hbm.at[pl.ds(off, CH)], idxd[k % _NSET])

    def wait_idx(k, off):
        pltpu.make_async_copy(src_hbm.at[pl.ds(off, CH)], idxs[k % _NSET],
                              semi).wait()
        pltpu.make_async_copy(dst_hbm.at[pl.ds(off, CH)], idxd[k % _NSET],
                              semi).wait()

    def fire_gather(k):
        pltpu.async_copy(u_hbm.at[idxs[k % _NSET]], rows[k % _NBUF], semg)

    def wait_gather(k):
        pltpu.make_async_copy(u_hbm.at[idxs[k % _NSET]], rows[k % _NBUF],
                              semg).wait()

    def fire_scatter(k):
        pltpu.async_copy(rows[k % _NBUF], acc.at[idxd[k % _NSET]], sems,
                         add=True)

    def wait_scatter(k):
        pltpu.make_async_copy(rows[k % _NBUF], acc.at[idxd[k % _NSET]],
                              sems).wait()

    pltpu.sync_copy(z_hbm, acc.at[pl.ds(t * RPT, RPT), :])
    for k0 in range(_LA):
        sync_idx(k0)
    for k0 in range(_LA, 8):
        fire_idx(k0, base + k0 * CH)
    plsc.subcore_barrier()
    for k0 in range(_LA):
        fire_gather(k0)

    def body(i, _):
        for u in range(_UNROLL):
            k = i * _UNROLL + u

            @pl.when(k >= 2)
            def _drain_scatter():
                wait_scatter(u + 8)

            @pl.when(k + _LA < NCHUNK)
            def _next_gather():
                wait_idx(u + _LA, base + (k + _LA) * CH)
                fire_gather(u + _LA)

            wait_gather(u)
            fire_scatter(u)

            @pl.when(k + 8 < NCHUNK)
            def _next_idx():
                fire_idx(u + 8, base + (k + 8) * CH)
        return _

    lax.fori_loop(0, NCHUNK // _UNROLL, body, None)
    wait_scatter(8)
    wait_scatter(9)
    plsc.subcore_barrier()

    @pl.when(t < NS - 1)
    def _copy_main():
        pltpu.sync_copy(acc.at[pl.ds(t * 624, 624), :],
                        out_hbm.at[pl.ds(c * N + t * 624, 624), :])

    @pl.when(t == NS - 1)
    def _copy_tail():
        pltpu.sync_copy(acc.at[pl.ds(624 * (NS - 1), 640), :],
                        out_hbm.at[pl.ds(c * N + 624 * (NS - 1), 640), :])


_mp_call = pl.kernel(
    _mp_body,
    out_type=jax.ShapeDtypeStruct((NC * N, D), _f32),
    mesh=_MESH,
    compiler_params=pltpu.CompilerParams(needs_layout_passes=False),
    scratch_types=[
        [pltpu.VMEM((CH,), jnp.int32) for _ in range(_NSET)],
        [pltpu.VMEM((CH,), jnp.int32) for _ in range(_NSET)],
        [pltpu.VMEM((CH, D), _f32) for _ in range(_NBUF)],
        pltpu.VMEM_SHARED((NPAD, D), _f32),
        pltpu.SemaphoreType.DMA,
        pltpu.SemaphoreType.DMA,
        pltpu.SemaphoreType.DMA,
    ],
)


def _prep_tc(deg0_ref, deg1_ref, x_ref, u0_ref, dinv_ref):
    dinv = lax.rsqrt(deg0_ref[...] + deg1_ref[...] + 1.0)
    u0_ref[...] = x_ref[...] * dinv
    dinv_ref[...] = dinv


_prep_call = pl.pallas_call(
    _prep_tc,
    out_shape=(jax.ShapeDtypeStruct((N, D), _f32),
               jax.ShapeDtypeStruct((N, 1), _f32)),
)


def _bn_relu_tc(zz, g, be):
    mu = jnp.mean(zz, axis=0, keepdims=True)
    dz = zz - mu
    var = jnp.mean(dz * dz, axis=0, keepdims=True)
    return jnp.maximum(dz * lax.rsqrt(var + 1e-5) * g + be, 0.0)


def _layer_tc(m_ref, u_ref, dinv_ref, w_ref, b_ref, g_ref, be_ref, un_ref):
    dinv = dinv_ref[...]
    s = (m_ref[0] + m_ref[1] + u_ref[...]) * dinv
    zz = jnp.dot(s, w_ref[...], preferred_element_type=_f32) + b_ref[...]
    h = _bn_relu_tc(zz, g_ref[...], be_ref[...])
    un_ref[...] = h * dinv


_layer_call = pl.pallas_call(
    _layer_tc,
    out_shape=jax.ShapeDtypeStruct((N, D), _f32),
)


def _heads_tc(m_ref, u_ref, dinv_ref, batch_ref, w_ref, b_ref, g_ref, be_ref,
              fcw_ref, fcb_ref, out_ref, pre_s, s_s, icnt_s):
    j = pl.program_id(0)

    @pl.when(j == 0)
    def _init():
        dinv = dinv_ref[...]
        pre_s[...] = (m_ref[0] + m_ref[1] + u_ref[...]) * dinv
        gid = lax.broadcasted_iota(jnp.int32, (G, N), 0)
        sel = (gid == batch_ref[...]).astype(_f32)
        s_s[...] = sel
        cnt = jnp.sum(sel, axis=1, keepdims=True)
        icnt_s[...] = 1.0 / jnp.maximum(cnt, 1.0)

    zz = jnp.dot(pre_s[...], w_ref[0], preferred_element_type=_f32) + b_ref[0]
    h = _bn_relu_tc(zz, g_ref[0], be_ref[0])
    sums = jnp.dot(s_s[...], h, preferred_element_type=_f32)
    pooled = sums * icnt_s[...]
    logits = jnp.dot(pooled, fcw_ref[0], preferred_element_type=_f32) + fcb_ref[0]
    mx = jnp.max(logits, axis=1, keepdims=True)
    ex = jnp.exp(logits - mx)
    out_ref[0] = (logits - mx) - jnp.log(jnp.sum(ex, axis=1, keepdims=True))


_heads_call = pl.pallas_call(
    _heads_tc,
    grid=(NSPLITS,),
    in_specs=[
        pl.BlockSpec((NC, N, D), lambda j: (0, 0, 0)),
        pl.BlockSpec((N, D), lambda j: (0, 0)),
        pl.BlockSpec((N, 1), lambda j: (0, 0)),
        pl.BlockSpec((1, N), lambda j: (0, 0)),
        pl.BlockSpec((1, D, D), lambda j: (j, 0, 0)),
        pl.BlockSpec((1, 1, D), lambda j: (j, 0, 0)),
        pl.BlockSpec((1, 1, D), lambda j: (j, 0, 0)),
        pl.BlockSpec((1, 1, D), lambda j: (j, 0, 0)),
        pl.BlockSpec((1, D, 10), lambda j: (j, 0, 0)),
        pl.BlockSpec((1, 1, 10), lambda j: (j, 0, 0)),
    ],
    out_specs=pl.BlockSpec((1, G, 10), lambda j: (j, 0, 0)),
    out_shape=jax.ShapeDtypeStruct((NSPLITS, G, 10), _f32),
    scratch_shapes=[
        pltpu.VMEM((N, D), _f32),
        pltpu.VMEM((G, N), _f32),
        pltpu.VMEM((G, 1), _f32),
    ],
)


def kernel(x, edge_index, batch, shared_W, shared_b, shared_g, shared_be,
           main_W, main_b, main_g, main_be, main_fcW, main_fcb,
           aux_W, aux_b, aux_g, aux_be, aux_fcW, aux_fcb):
    epw_raw = E // NW
    src = edge_index[0].reshape(NW, epw_raw)
    dst = edge_index[1].reshape(NW, epw_raw)
    srcp = jnp.pad(src, ((0, 0), (0, EPW - epw_raw))).reshape(-1)
    dstp = jnp.pad(dst, ((0, 0), (0, EPW - epw_raw)), constant_values=N).reshape(-1)
    z_rows = jnp.zeros((RPT, D), _f32)

    deg = _deg_call(dstp)
    deg0 = deg[:N].reshape(N, 1)
    deg1 = deg[NPAD:NPAD + N].reshape(N, 1)
    u0, dinv = _prep_call(deg0, deg1, x)

    m0 = _mp_call(u0, srcp, dstp, z_rows).reshape(NC, N, D)
    u1 = _layer_call(m0, u0, dinv, shared_W[0], shared_b[0].reshape(1, D),
                     shared_g[0].reshape(1, D), shared_be[0].reshape(1, D))
    m1 = _mp_call(u1, srcp, dstp, z_rows).reshape(NC, N, D)
    u2 = _layer_call(m1, u1, dinv, shared_W[1], shared_b[1].reshape(1, D),
                     shared_g[1].reshape(1, D), shared_be[1].reshape(1, D))
    m2 = _mp_call(u2, srcp, dstp, z_rows).reshape(NC, N, D)

    window = D // NSPLITS
    Ws = jnp.stack([main_W] + [jnp.roll(aux_W[i], window * (i + 1), axis=0)
                               for i in range(NSPLITS - 1)])
    bs = jnp.concatenate([main_b[None], aux_b]).reshape(NSPLITS, 1, D)
    gs = jnp.concatenate([main_g[None], aux_g]).reshape(NSPLITS, 1, D)
    bes = jnp.concatenate([main_be[None], aux_be]).reshape(NSPLITS, 1, D)
    fcWs = jnp.concatenate([main_fcW[None], aux_fcW])
    fcbs = jnp.concatenate([main_fcb[None], aux_fcb]).reshape(NSPLITS, 1, 10)

    outs = _heads_call(m2, u2, dinv, batch.reshape(1, N),
                       Ws, bs, gs, bes, fcWs, fcbs)
    return (outs[0], jnp.swapaxes(outs[1:], 0, 1))


# speedup vs baseline: 2.4454x; 2.2409x over previous
import jax
import jax.numpy as jnp
from jax import lax
from jax.experimental import pallas as pl
from jax.experimental.pallas import tpu as pltpu
from jax.experimental.pallas import tpu_sc as plsc

N = 10000
D = 128
E = 320000
G = 64
NSPLITS = 9
NC = 2
NS = 16
NW = NC * NS
EPW = 10240
CHUNKS = EPW // 128
NPAD = 10240
RPT = NPAD // NS

_f32 = jnp.float32
_MESH = plsc.VectorSubcoreMesh(core_axis_name="c", subcore_axis_name="s")


TROWS = N // NC
EPR = E // NS
CAP = 20672
CNTPAD = 192


def _degp_body(src_hbm, dst_hbm, deg_out, psrc_hbm, pdst_hbm, cnt_hbm,
               srcv, dstv, psrcv, pdstv, hist, tbuf, accb, cntv, partials):
    c = lax.axis_index("c")
    t = lax.axis_index("s")
    w = c * NS + t
    lo0 = c * TROWS
    z16 = jnp.zeros((16,), _f32)
    ones16 = jnp.ones((16,), _f32)

    def zero_hist(i, _):
        hist[pl.ds(i * 16, 16)] = z16
        return _

    lax.fori_loop(0, NPAD // 16, zero_hist, None)
    pltpu.sync_copy(src_hbm.at[pl.ds(t * EPR, EPR)], srcv)
    pltpu.sync_copy(dst_hbm.at[pl.ds(t * EPR, EPR)], dstv)

    def scan(j, pos):
        s16 = srcv[pl.ds(j * 16, 16)]
        d16 = dstv[pl.ds(j * 16, 16)]
        plsc.addupdate_scatter(hist, [d16], ones16)
        lo = s16 - lo0
        m = (lo >= 0) & (lo < TROWS)
        plsc.store_compressed(psrcv.at[pl.ds(pos, 16)], lo, mask=m)
        plsc.store_compressed(pdstv.at[pl.ds(pos, 16)], d16, mask=m)
        return pos + jnp.max(plsc.all_reduce_population_count(m), axis=0)

    pos = lax.fori_loop(0, EPR // 16, scan, jnp.int32(0))
    cnt = ((pos + CNTPAD - 1) // CNTPAD) * CNTPAD

    def pad(r, _):
        psrcv[pl.ds(pos + r * 16, 16)] = jnp.zeros((16,), jnp.int32)
        pdstv[pl.ds(pos + r * 16, 16)] = jnp.full((16,), N, jnp.int32)
        return _

    lax.fori_loop(0, (cnt - pos + 15) // 16, pad, None)
    cntv[...] = jnp.full((16,), 0, jnp.int32) + cnt
    pltpu.sync_copy(psrcv, psrc_hbm.at[pl.ds(w * CAP, CAP)])
    pltpu.sync_copy(pdstv, pdst_hbm.at[pl.ds(w * CAP, CAP)])
    pltpu.sync_copy(cntv, cnt_hbm.at[pl.ds(w * 16, 16)])

    pltpu.sync_copy(hist, partials.at[pl.ds(t * NPAD, NPAD)])
    plsc.subcore_barrier()

    def zero_accb(i, _):
        accb[pl.ds(i * 16, 16)] = z16
        return _

    lax.fori_loop(0, RPT // 16, zero_accb, None)
    for tt in range(NS):
        pltpu.sync_copy(partials.at[pl.ds(tt * NPAD + t * RPT, RPT)], tbuf)

        def addv(i, _):
            accb[pl.ds(i * 16, 16)] = accb[pl.ds(i * 16, 16)] + tbuf[pl.ds(i * 16, 16)]
            return _

        lax.fori_loop(0, RPT // 16, addv, None)
    pltpu.sync_copy(accb, deg_out.at[pl.ds(c * NPAD + t * RPT, RPT)])


_degp_call = pl.kernel(
    _degp_body,
    out_type=(jax.ShapeDtypeStruct((NC * NPAD,), _f32),
              jax.ShapeDtypeStruct((NW * CAP,), jnp.int32),
              jax.ShapeDtypeStruct((NW * CAP,), jnp.int32),
              jax.ShapeDtypeStruct((NW * 16,), jnp.int32)),
    mesh=_MESH,
    compiler_params=pltpu.CompilerParams(needs_layout_passes=False),
    scratch_types=[
        pltpu.VMEM((EPR,), jnp.int32),
        pltpu.VMEM((EPR,), jnp.int32),
        pltpu.VMEM((CAP,), jnp.int32),
        pltpu.VMEM((CAP,), jnp.int32),
        pltpu.VMEM((NPAD,), _f32),
        pltpu.VMEM((RPT,), _f32),
        pltpu.VMEM((RPT,), _f32),
        pltpu.VMEM((16,), jnp.int32),
        pltpu.VMEM_SHARED((NS * NPAD,), _f32),
    ],
)


CH = 32
_NSET = 6
_UNROLL = 6
MPAD = 10112
MRPT = MPAD // NS


def _mp_body(u_hbm, src_hbm, dst_hbm, cnt_hbm, z_hbm, out_hbm,
             idxs, idxd, rows, cntv, table, acc, semi, semg, sems):
    c = lax.axis_index("c")
    t = lax.axis_index("s")
    w = c * NS + t
    base = w * CAP
    lo0 = c * TROWS

    def fire_idx(u, off):
        pltpu.async_copy(src_hbm.at[pl.ds(off, CH)], idxs[u % _NSET], semi)
        pltpu.async_copy(dst_hbm.at[pl.ds(off, CH)], idxd[u % _NSET], semi)

    def wait_idx(u, off):
        pltpu.make_async_copy(src_hbm.at[pl.ds(off, CH)], idxs[u % _NSET],
                              semi).wait()
        pltpu.make_async_copy(dst_hbm.at[pl.ds(off, CH)], idxd[u % _NSET],
                              semi).wait()

    def fire_gather(u):
        pltpu.async_copy(table.at[idxs[u % _NSET]], rows[u % 2], semg)

    def wait_gather(u):
        pltpu.make_async_copy(table.at[idxs[u % _NSET]], rows[u % 2],
                              semg).wait()

    def fire_scatter(u):
        pltpu.async_copy(rows[u % 2], acc.at[idxd[u % _NSET]], sems, add=True)

    def wait_scatter(u):
        pltpu.make_async_copy(rows[u % 2], acc.at[idxd[u % _NSET]],
                              sems).wait()

    @pl.when(t < NS - 1)
    def _stage_main():
        pltpu.sync_copy(u_hbm.at[pl.ds(lo0 + t * 312, 312), :],
                        table.at[pl.ds(t * 312, 312), :])

    @pl.when(t == NS - 1)
    def _stage_tail():
        pltpu.sync_copy(u_hbm.at[pl.ds(lo0 + 312 * (NS - 1), 320), :],
                        table.at[pl.ds(312 * (NS - 1), 320), :])

    pltpu.sync_copy(z_hbm, acc.at[pl.ds(t * MRPT, MRPT), :])
    pltpu.sync_copy(cnt_hbm.at[pl.ds(w * 16, 16)], cntv)
    nch = jnp.max(cntv[...], axis=0) // CH
    nsup = nch // _UNROLL
    plsc.subcore_barrier()

    @pl.when(nsup > 0)
    def _prologue():
        pltpu.sync_copy(src_hbm.at[pl.ds(base, CH)], idxs[0])
        pltpu.sync_copy(dst_hbm.at[pl.ds(base, CH)], idxd[0])
        fire_gather(0)
        for k0 in range(1, 5):
            fire_idx(k0, base + k0 * CH)

    def body(i, _):
        for u in range(_UNROLL):
            k = i * _UNROLL + u
            wait_gather(u)
            fire_scatter(u)

            @pl.when(k + 1 < nch)
            def _next_gather():
                @pl.when(k >= 1)
                def _drain():
                    wait_scatter(u + 5)
                wait_idx(u + 1, base + (k + 1) * CH)
                fire_gather(u + 1)

            @pl.when(k + 5 < nch)
            def _next_idx():
                fire_idx(u + 5, base + (k + 5) * CH)
        return _

    lax.fori_loop(0, nsup, body, None)

    @pl.when(nsup > 0)
    def _tail():
        wait_scatter(5)

    plsc.subcore_barrier()

    @pl.when(t < NS - 1)
    def _copy_main():
        pltpu.sync_copy(acc.at[pl.ds(t * 624, 624), :],
                        out_hbm.at[pl.ds(c * N + t * 624, 624), :])

    @pl.when(t == NS - 1)
    def _copy_tail():
        pltpu.sync_copy(acc.at[pl.ds(624 * (NS - 1), 640), :],
                        out_hbm.at[pl.ds(c * N + 624 * (NS - 1), 640), :])


_mp_call = pl.kernel(
    _mp_body,
    out_type=jax.ShapeDtypeStruct((NC * N, D), _f32),
    mesh=_MESH,
    compiler_params=pltpu.CompilerParams(needs_layout_passes=False),
    scratch_types=[
        [pltpu.VMEM((CH,), jnp.int32) for _ in range(_NSET)],
        [pltpu.VMEM((CH,), jnp.int32) for _ in range(_NSET)],
        [pltpu.VMEM((CH, D), _f32) for _ in range(2)],
        pltpu.VMEM((16,), jnp.int32),
        pltpu.VMEM_SHARED((TROWS, D), _f32),
        pltpu.VMEM_SHARED((MPAD, D), _f32),
        pltpu.SemaphoreType.DMA,
        pltpu.SemaphoreType.DMA,
        pltpu.SemaphoreType.DMA,
    ],
)


def _prep_tc(deg_ref, x_ref, u0_ref, dinv_ref):
    dinv = lax.rsqrt(deg_ref[...] + 1.0)
    u0_ref[...] = x_ref[...] * dinv
    dinv_ref[...] = dinv


_prep_call = pl.pallas_call(
    _prep_tc,
    out_shape=(jax.ShapeDtypeStruct((N, D), _f32),
               jax.ShapeDtypeStruct((N, 1), _f32)),
)


def _bn_relu_tc(zz, g, be):
    mu = jnp.mean(zz, axis=0, keepdims=True)
    dz = zz - mu
    var = jnp.mean(dz * dz, axis=0, keepdims=True)
    return jnp.maximum(dz * lax.rsqrt(var + 1e-5) * g + be, 0.0)


def _layer_tc(m_ref, u_ref, dinv_ref, w_ref, b_ref, g_ref, be_ref, un_ref):
    dinv = dinv_ref[...]
    s = (m_ref[0] + m_ref[1] + u_ref[...]) * dinv
    zz = jnp.dot(s, w_ref[...], preferred_element_type=_f32) + b_ref[...]
    h = _bn_relu_tc(zz, g_ref[...], be_ref[...])
    un_ref[...] = h * dinv


_layer_call = pl.pallas_call(
    _layer_tc,
    out_shape=jax.ShapeDtypeStruct((N, D), _f32),
)


def _heads_tc(m_ref, u_ref, dinv_ref, batch_ref, w_ref, b_ref, g_ref, be_ref,
              fcw_ref, fcb_ref, out_ref, pre_s, s_s, icnt_s):
    j = pl.program_id(0)

    @pl.when(j == 0)
    def _init():
        dinv = dinv_ref[...]
        pre_s[...] = (m_ref[0] + m_ref[1] + u_ref[...]) * dinv
        gid = lax.broadcasted_iota(jnp.int32, (G, N), 0)
        sel = (gid == batch_ref[...]).astype(_f32)
        s_s[...] = sel
        cnt = jnp.sum(sel, axis=1, keepdims=True)
        icnt_s[...] = 1.0 / jnp.maximum(cnt, 1.0)

    zz = jnp.dot(pre_s[...], w_ref[0], preferred_element_type=_f32) + b_ref[0]
    h = _bn_relu_tc(zz, g_ref[0], be_ref[0])
    sums = jnp.dot(s_s[...], h, preferred_element_type=_f32)
    pooled = sums * icnt_s[...]
    logits = jnp.dot(pooled, fcw_ref[0], preferred_element_type=_f32) + fcb_ref[0]
    mx = jnp.max(logits, axis=1, keepdims=True)
    ex = jnp.exp(logits - mx)
    out_ref[0] = (logits - mx) - jnp.log(jnp.sum(ex, axis=1, keepdims=True))


_heads_call = pl.pallas_call(
    _heads_tc,
    grid=(NSPLITS,),
    in_specs=[
        pl.BlockSpec((NC, N, D), lambda j: (0, 0, 0)),
        pl.BlockSpec((N, D), lambda j: (0, 0)),
        pl.BlockSpec((N, 1), lambda j: (0, 0)),
        pl.BlockSpec((1, N), lambda j: (0, 0)),
        pl.BlockSpec((1, D, D), lambda j: (j, 0, 0)),
        pl.BlockSpec((1, 1, D), lambda j: (j, 0, 0)),
        pl.BlockSpec((1, 1, D), lambda j: (j, 0, 0)),
        pl.BlockSpec((1, 1, D), lambda j: (j, 0, 0)),
        pl.BlockSpec((1, D, 10), lambda j: (j, 0, 0)),
        pl.BlockSpec((1, 1, 10), lambda j: (j, 0, 0)),
    ],
    out_specs=pl.BlockSpec((1, G, 10), lambda j: (j, 0, 0)),
    out_shape=jax.ShapeDtypeStruct((NSPLITS, G, 10), _f32),
    scratch_shapes=[
        pltpu.VMEM((N, D), _f32),
        pltpu.VMEM((G, N), _f32),
        pltpu.VMEM((G, 1), _f32),
    ],
)


def kernel(x, edge_index, batch, shared_W, shared_b, shared_g, shared_be,
           main_W, main_b, main_g, main_be, main_fcW, main_fcb,
           aux_W, aux_b, aux_g, aux_be, aux_fcW, aux_fcb):
    z_rows = jnp.zeros((MRPT, D), _f32)

    deg, psrc, pdst, counts = _degp_call(edge_index[0], edge_index[1])
    deg_col = deg[:N].reshape(N, 1)
    u0, dinv = _prep_call(deg_col, x)

    m0 = _mp_call(u0, psrc, pdst, counts, z_rows).reshape(NC, N, D)
    u1 = _layer_call(m0, u0, dinv, shared_W[0], shared_b[0].reshape(1, D),
                     shared_g[0].reshape(1, D), shared_be[0].reshape(1, D))
    m1 = _mp_call(u1, psrc, pdst, counts, z_rows).reshape(NC, N, D)
    u2 = _layer_call(m1, u1, dinv, shared_W[1], shared_b[1].reshape(1, D),
                     shared_g[1].reshape(1, D), shared_be[1].reshape(1, D))
    m2 = _mp_call(u2, psrc, pdst, counts, z_rows).reshape(NC, N, D)

    window = D // NSPLITS
    Ws = jnp.stack([main_W] + [jnp.roll(aux_W[i], window * (i + 1), axis=0)
                               for i in range(NSPLITS - 1)])
    bs = jnp.concatenate([main_b[None], aux_b]).reshape(NSPLITS, 1, D)
    gs = jnp.concatenate([main_g[None], aux_g]).reshape(NSPLITS, 1, D)
    bes = jnp.concatenate([main_be[None], aux_be]).reshape(NSPLITS, 1, D)
    fcWs = jnp.concatenate([main_fcW[None], aux_fcW])
    fcbs = jnp.concatenate([main_fcb[None], aux_fcb]).reshape(NSPLITS, 1, 10)

    outs = _heads_call(m2, u2, dinv, batch.reshape(1, N),
                       Ws, bs, gs, bes, fcWs, fcbs)
    return (outs[0], jnp.swapaxes(outs[1:], 0, 1))
